# baseline (speedup 1.0000x reference)
import functools

import jax
import jax.numpy as jnp
from jax import lax
from jax.experimental import pallas as pl
from jax.experimental.pallas import tpu as pltpu
from jax.experimental.pallas import tpu_sc as plsc

_N = 10000
_E = 320000
_D = 128
_H = 8
_C1 = 64
_C2 = 64
_NL = 40

_NTILES = 32
_R = 157
_NSTR = 64
_NP = _R * _NSTR
_BE = 1600
_NBLK = _E // _BE
_SCAP = _E + 8192
_WIN = 2048
_W1W = _H * _C1
_MASKHI = -65536

_GDN = lax.GatherDimensionNumbers(
    offset_dims=(), collapsed_slice_dims=(0,), start_index_map=(0,))


def _lane(v, i):
    idx = jnp.full((16, 1), i, dtype=jnp.int32)
    return lax.gather(v, idx, dimension_numbers=_GDN, slice_sizes=(1,),
                      mode=lax.GatherScatterMode.PROMISE_IN_BOUNDS)


def _mesh():
    return plsc.VectorSubcoreMesh(core_axis_name="c", subcore_axis_name="s",
                                  num_cores=2, num_subcores=16)


def _wid():
    return lax.axis_index("s") * 2 + lax.axis_index("c")


def _match_body(pk_h, src_h, dst_h, msrc_h, mdst_h, mee_h, cnt_h,
                ptab, srcb, dstb, msrcb, mdstb, meeb, cntb):
    wid = _wid()
    iota = lax.iota(jnp.int32, 16)
    pltpu.sync_copy(pk_h, ptab)
    for sw in range(2):
        sid = sw * _NTILES + wid
        lo = sid * _R

        def blk(b, off16, lo=lo, sid=sid):
            off = off16 * 16
            eb = b * _BE
            pltpu.sync_copy(src_h.at[pl.ds(eb, _BE)], srcb)
            pltpu.sync_copy(dst_h.at[pl.ds(eb, _BE)], dstb)

            def grp(g, cur):
                o16 = g * 16
                s16 = srcb[pl.ds(o16, 16)]
                d16 = dstb[pl.ds(o16, 16)]
                dr = d16 - lo
                m = (dr >= 0) & (dr < _R)
                plsc.store_compressed(msrcb.at[pl.ds(cur, 16)], s16, mask=m)
                plsc.store_compressed(mdstb.at[pl.ds(cur, 16)], dr, mask=m)
                return cur + jnp.max(plsc.all_reduce_population_count(m))
            kb = lax.fori_loop(0, _BE // 16, grp, jnp.int32(0))
            msrcb[pl.ds(kb, 16)] = jnp.zeros((16,), jnp.int32)
            mdstb[pl.ds(kb, 16)] = jnp.full((16,), _R, jnp.int32)
            ngb = (kb + 15) // 16

            def ee(j, _):
                jo = j * 16
                s16 = msrcb[pl.ds(jo, 16)]
                d16 = jnp.minimum(mdstb[pl.ds(jo, 16)] + lo, _N - 1)
                erel = jo + iota
                for h in range(_H):
                    pd = plsc.load_gather(ptab, [d16 * _H + h])
                    ps = plsc.load_gather(ptab, [s16 * _H + h])
                    a_self = plsc.bitcast(pd & _MASKHI, jnp.float32)
                    a_nei = plsc.bitcast(ps << 16, jnp.float32)
                    e = a_self + a_nei
                    e = jnp.where(e >= 0.0, e, 0.2 * e)
                    plsc.store_scatter(meeb, [erel * _H + h], jnp.exp(e))
                return 0
            lax.fori_loop(0, ngb, ee, 0)
            for t in range(_H):
                meeb[pl.ds(kb * _H + t * 16, 16)] = jnp.zeros(
                    (16,), jnp.float32)

            def flush(j, _, sid=sid):
                jo = j * 16
                so = sid * _SCAP + off + jo
                pltpu.sync_copy(msrcb.at[pl.ds(jo, 16)],
                                msrc_h.at[pl.ds(so, 16)])
                pltpu.sync_copy(mdstb.at[pl.ds(jo, 16)],
                                mdst_h.at[pl.ds(so, 16)])
                pltpu.sync_copy(meeb.at[pl.ds(jo * _H, 16 * _H)],
                                mee_h.at[pl.ds(so * _H, 16 * _H)])
                return 0
            lax.fori_loop(0, ngb, flush, 0)
            return off16 + ngb
        total16 = lax.fori_loop(0, _NBLK, blk, jnp.int32(0))
        cntb[pl.ds(0, 16)] = jnp.broadcast_to(total16 * 16, (16,))
        pltpu.sync_copy(cntb, cnt_h.at[pl.ds(sid * 16, 16)])


def _sc_match(pk1, src, dst):
    f = pl.kernel(
        _match_body,
        out_type=(jax.ShapeDtypeStruct((_NSTR * _SCAP,), jnp.int32),
                  jax.ShapeDtypeStruct((_NSTR * _SCAP,), jnp.int32),
                  jax.ShapeDtypeStruct((_NSTR * _SCAP * _H,), jnp.float32),
                  jax.ShapeDtypeStruct((_NSTR * 16,), jnp.int32)),
        mesh=_mesh(),
        compiler_params=pltpu.CompilerParams(needs_layout_passes=False),
        scratch_types=[
            pltpu.VMEM((_N * _H,), jnp.int32),
            pltpu.VMEM((_BE,), jnp.int32),
            pltpu.VMEM((_BE,), jnp.int32),
            pltpu.VMEM((_BE + 16,), jnp.int32),
            pltpu.VMEM((_BE + 16,), jnp.int32),
            pltpu.VMEM(((_BE + 16) * _H,), jnp.float32),
            pltpu.VMEM((16,), jnp.int32),
        ],
    )
    return f(pk1, src, dst)


def _agg1_body(msrc_h, mdst_h, mee_h, cnt_h, feat_h, out_h,
               msrcw, mdstw, meew, fbuf, accum, den, cntb, sem):
    wid = _wid()
    iota = lax.iota(jnp.int32, 16)
    zero16 = jnp.zeros((16,), jnp.float32)
    for sw in range(2):
        sid = sw * _NTILES + wid
        lo = sid * _R
        pltpu.sync_copy(cnt_h.at[pl.ds(sid * 16, 16)], cntb)
        kpad = jnp.max(cntb[pl.ds(0, 16)])

        def zrow(r, _):
            for cc in range(_W1W // 16):
                accum[pl.ds(r * _W1W + cc * 16, 16)] = zero16
            return 0
        lax.fori_loop(0, _R + 1, zrow, 0)

        def zden(i, _):
            den[pl.ds(i * 16, 16)] = zero16
            return 0
        lax.fori_loop(0, (_R + 1) * _H // 16, zden, 0)

        nwin = (kpad + _WIN - 1) // _WIN

        def win(wb, _, sid=sid):
            wo = wb * _WIN
            so = sid * _SCAP + wo
            pltpu.sync_copy(msrc_h.at[pl.ds(so, _WIN)], msrcw)
            pltpu.sync_copy(mdst_h.at[pl.ds(so, _WIN)], mdstw)
            pltpu.sync_copy(mee_h.at[pl.ds(so * _H, _WIN * _H)],
                            meew.at[pl.ds(0, _WIN * _H)])
            ngw = jnp.minimum(_WIN // 16, (kpad - wo) // 16)

            def proc(j, _):
                jo = j * 16
                pltpu.async_copy(feat_h.at[msrcw.at[pl.ds(jo, 16)]], fbuf,
                                 sem).wait()
                md = mdstw[pl.ds(jo, 16)]

                def edge(g2, _2):
                    db = _lane(md, g2)
                    eerow = meew[pl.ds((jo + g2) * _H, 16)]
                    plsc.addupdate_scatter(den, [db * _H + iota], eerow,
                                           mask=iota < _H)
                    dbase = db * _W1W
                    for h in range(_H):
                        w = _lane(eerow, h)
                        for cc in range(4):
                            o = h * _C1 + cc * 16
                            v = fbuf[g2, pl.ds(o, 16)]
                            plsc.addupdate_scatter(accum, [dbase + o + iota],
                                                   w * v)
                    return 0
                lax.fori_loop(0, 16, edge, 0)
                return 0
            lax.fori_loop(0, ngw, proc, 0)
            return 0
        lax.fori_loop(0, nwin, win, 0)

        def drow(r2, _):
            d16 = den[pl.ds(r2 * 16, 16)]
            rec = 1.0 / (d16 + 1e-9)
            for rr in range(2):
                row = r2 * 2 + rr
                for h in range(_H):
                    w = _lane(rec, rr * _H + h)
                    for cc in range(4):
                        o = row * _W1W + h * _C1 + cc * 16
                        accum[pl.ds(o, 16)] = accum[pl.ds(o, 16)] * w
            return 0
        lax.fori_loop(0, (_R + 1) // 2, drow, 0)
        pltpu.sync_copy(accum.at[pl.ds(0, _R * _W1W)],
                        out_h.at[pl.ds(lo * _W1W, _R * _W1W)])


def _sc_agg1(msrc_s, mdst_s, mee_s, cnts, feat1):
    f = pl.kernel(
        _agg1_body,
        out_type=jax.ShapeDtypeStruct((_NP * _W1W,), jnp.float32),
        mesh=_mesh(),
        compiler_params=pltpu.CompilerParams(needs_layout_passes=False),
        scratch_types=[
            pltpu.VMEM((_WIN,), jnp.int32),
            pltpu.VMEM((_WIN,), jnp.int32),
            pltpu.VMEM((_WIN * _H + 16,), jnp.float32),
            pltpu.VMEM((16, _W1W), jnp.float32),
            pltpu.VMEM(((_R + 1) * _W1W,), jnp.float32),
            pltpu.VMEM(((_R + 1) * _H,), jnp.float32),
            pltpu.VMEM((16,), jnp.int32),
            pltpu.SemaphoreType.DMA,
        ],
    )
    return f(msrc_s, mdst_s, mee_s, cnts, feat1)


def _agg2_body(msrc_h, mdst_h, cnt_h, pk_h, feat_h, out_h,
               ptab, msrcw, mdstw, fbuf, accum, den, cntb, sem):
    wid = _wid()
    iota = lax.iota(jnp.int32, 16)
    zero16 = jnp.zeros((16,), jnp.float32)
    pltpu.sync_copy(pk_h, ptab)
    for sw in range(2):
        sid = sw * _NTILES + wid
        lo = sid * _R
        pltpu.sync_copy(cnt_h.at[pl.ds(sid * 16, 16)], cntb)
        kpad = jnp.max(cntb[pl.ds(0, 16)])

        def zrow(r, _):
            for cc in range(_C2 // 16):
                accum[pl.ds(r * _C2 + cc * 16, 16)] = zero16
            return 0
        lax.fori_loop(0, _R + 3, zrow, 0)

        def zden(i, _):
            den[pl.ds(i * 16, 16)] = zero16
            return 0
        lax.fori_loop(0, (_R + 3) // 16, zden, 0)

        nwin = (kpad + _WIN - 1) // _WIN

        def win(wb, _, sid=sid, lo=lo):
            wo = wb * _WIN
            so = sid * _SCAP + wo
            pltpu.sync_copy(msrc_h.at[pl.ds(so, _WIN)], msrcw)
            pltpu.sync_copy(mdst_h.at[pl.ds(so, _WIN)], mdstw)
            ngw = jnp.minimum(_WIN // 16, (kpad - wo) // 16)

            def proc(j, _):
                jo = j * 16
                pltpu.async_copy(feat_h.at[msrcw.at[pl.ds(jo, 16)]], fbuf,
                                 sem).wait()
                md = mdstw[pl.ds(jo, 16)]
                s16 = msrcw[pl.ds(jo, 16)]
                pd = plsc.load_gather(ptab, [jnp.minimum(md + lo, _N - 1)])
                ps = plsc.load_gather(ptab, [s16])
                a_self = plsc.bitcast(pd & _MASKHI, jnp.float32)
                a_nei = plsc.bitcast(ps << 16, jnp.float32)
                e = a_self + a_nei
                e = jnp.where(e >= 0.0, e, 0.2 * e)
                w16 = jnp.where(md < _R, jnp.exp(e), 0.0)

                def edge(g2, _2):
                    w = _lane(w16, g2)
                    db = _lane(md, g2)
                    plsc.addupdate_scatter(den, [db], w, mask=iota == 0)
                    dbase = db * _C2
                    for cc in range(4):
                        o = cc * 16
                        v = fbuf[g2, pl.ds(o, 16)]
                        plsc.addupdate_scatter(accum, [dbase + o + iota],
                                               w * v)
                    return 0
                lax.fori_loop(0, 16, edge, 0)
                return 0
            lax.fori_loop(0, ngw, proc, 0)
            return 0
        lax.fori_loop(0, nwin, win, 0)

        def drow(r16, _):
            d16 = den[pl.ds(r16 * 16, 16)]
            rec = 1.0 / (d16 + 1e-9)
            for rr in range(16):
                row = r16 * 16 + rr
                w = _lane(rec, rr)
                for cc in range(4):
                    o = row * _C2 + cc * 16
                    accum[pl.ds(o, 16)] = accum[pl.ds(o, 16)] * w
            return 0
        lax.fori_loop(0, (_R + 3) // 16, drow, 0)
        pltpu.sync_copy(accum.at[pl.ds(0, _R * _C2)],
                        out_h.at[pl.ds(lo * _C2, _R * _C2)])


def _sc_agg2(msrc_s, mdst_s, cnts, pk2, feat2):
    f = pl.kernel(
        _agg2_body,
        out_type=jax.ShapeDtypeStruct((_NP * _C2,), jnp.float32),
        mesh=_mesh(),
        compiler_params=pltpu.CompilerParams(needs_layout_passes=False),
        scratch_types=[
            pltpu.VMEM((_N,), jnp.int32),
            pltpu.VMEM((_WIN,), jnp.int32),
            pltpu.VMEM((_WIN,), jnp.int32),
            pltpu.VMEM((16, 128), jnp.float32),
            pltpu.VMEM(((_R + 3) * _C2,), jnp.float32),
            pltpu.VMEM((_R + 3,), jnp.float32),
            pltpu.VMEM((16,), jnp.int32),
            pltpu.SemaphoreType.DMA,
        ],
    )
    return f(msrc_s, mdst_s, cnts, pk2, feat2)


_BM = 400


def _tca_body(x_ref, w_ref, a_ref, f_ref, aux_ref):
    f = jnp.dot(x_ref[...], w_ref[...], preferred_element_type=jnp.float32)
    f_ref[...] = f
    aux_ref[...] = jnp.dot(f, a_ref[...], preferred_element_type=jnp.float32)


def _tc_a(x, w1r, acmb):
    return pl.pallas_call(
        _tca_body,
        grid=(_N // _BM,),
        in_specs=[pl.BlockSpec((_BM, _D), lambda i: (i, 0)),
                  pl.BlockSpec((_D, _W1W), lambda i: (0, 0)),
                  pl.BlockSpec((_W1W, 128), lambda i: (0, 0))],
        out_specs=[pl.BlockSpec((_BM, _W1W), lambda i: (i, 0)),
                   pl.BlockSpec((_BM, 128), lambda i: (i, 0))],
        out_shape=[jax.ShapeDtypeStruct((_N, _W1W), jnp.float32),
                   jax.ShapeDtypeStruct((_N, 128), jnp.float32)],
    )(x, w1r, acmb)


def _tcb_body(o1_ref, b1_ref, w2_ref, a2_ref, f2_ref, aux2_ref):
    v = o1_ref[...] + b1_ref[...]
    h1 = jnp.where(v > 0.0, v, jnp.exp(v) - 1.0)
    f2 = jnp.dot(h1, w2_ref[...], preferred_element_type=jnp.float32)
    f2_ref[...] = f2
    aux2_ref[...] = jnp.dot(f2, a2_ref[...],
                            preferred_element_type=jnp.float32)


def _tc_b(o1, b1r, w2r, a2cmb):
    return pl.pallas_call(
        _tcb_body,
        grid=(_N // _BM,),
        in_specs=[pl.BlockSpec((_BM, _W1W), lambda i: (i, 0)),
                  pl.BlockSpec((1, _W1W), lambda i: (0, 0)),
                  pl.BlockSpec((_W1W, 128), lambda i: (0, 0)),
                  pl.BlockSpec((128, 128), lambda i: (0, 0))],
        out_specs=[pl.BlockSpec((_BM, 128), lambda i: (i, 0)),
                   pl.BlockSpec((_BM, 128), lambda i: (i, 0))],
        out_shape=[jax.ShapeDtypeStruct((_N, 128), jnp.float32),
                   jax.ShapeDtypeStruct((_N, 128), jnp.float32)],
    )(o1, b1r, w2r, a2cmb)


def _tcc_body(o2_ref, b2_ref, wd_ref, bd_ref, g_ref, l_ref):
    i = pl.program_id(0)
    v = o2_ref[...] + b2_ref[...]
    h2 = jnp.where(v > 0.0, v, jnp.exp(v) - 1.0)
    ps = jnp.sum(h2, axis=0, keepdims=True)

    @pl.when(i == 0)
    def _():
        g_ref[...] = ps

    @pl.when(i > 0)
    def _():
        g_ref[...] = g_ref[...] + ps

    @pl.when(i == pl.num_programs(0) - 1)
    def _():
        l = jnp.dot(g_ref[...], wd_ref[...],
                    preferred_element_type=jnp.float32) + bd_ref[...]
        m = jnp.max(l, axis=-1, keepdims=True)
        z = jnp.exp(l - m)
        l_ref[...] = z / jnp.sum(z, axis=-1, keepdims=True)


def _tc_c(o2, b2r, wdp, bdp):
    bm = 400
    return pl.pallas_call(
        _tcc_body,
        grid=(_N // bm,),
        in_specs=[pl.BlockSpec((bm, _C2), lambda i: (i, 0)),
                  pl.BlockSpec((1, _C2), lambda i: (0, 0)),
                  pl.BlockSpec((_C2, 128), lambda i: (0, 0)),
                  pl.BlockSpec((1, 128), lambda i: (0, 0))],
        out_specs=[pl.BlockSpec((1, _C2), lambda i: (0, 0)),
                   pl.BlockSpec((1, 128), lambda i: (0, 0))],
        out_shape=[jax.ShapeDtypeStruct((1, _C2), jnp.float32),
                   jax.ShapeDtypeStruct((1, 128), jnp.float32)],
    )(o2, b2r, wdp, bdp)


def _pack(a_hi, a_lo):
    hi = lax.bitcast_convert_type(a_hi.astype(jnp.bfloat16),
                                  jnp.uint16).astype(jnp.uint32) << 16
    lo = lax.bitcast_convert_type(a_lo.astype(jnp.bfloat16),
                                  jnp.uint16).astype(jnp.uint32)
    return lax.bitcast_convert_type(hi | lo, jnp.int32)


def kernel(x, edge_index, W1, a_src1, a_dst1, b1, W2, a_src2, a_dst2, b2,
           Wd, bd):
    src = edge_index[0]
    dst = edge_index[1]
    w1r = W1.reshape(_D, _W1W)
    eye8 = jnp.eye(_H, dtype=jnp.float32)
    acmb = jnp.concatenate([
        jnp.einsum("hc,hk->hck", a_src1, eye8).reshape(_W1W, _H),
        jnp.einsum("hc,hk->hck", a_dst1, eye8).reshape(_W1W, _H),
    ], axis=1)
    acmb = jnp.pad(acmb, ((0, 0), (0, 128 - 2 * _H)))
    w2r = jnp.pad(W2.reshape(_W1W, _C2), ((0, 0), (0, 128 - _C2)))
    a2cmb = jnp.pad(jnp.concatenate([a_src2.T, a_dst2.T], axis=1),
                    ((0, 128 - _C2), (0, 126)))
    wdp = jnp.pad(Wd, ((0, 0), (0, 128 - _NL)))
    bdp = jnp.concatenate(
        [bd, jnp.full((128 - _NL,), -1e30, jnp.float32)]).reshape(1, 128)

    feat1, aux1 = _tc_a(x, w1r, acmb)
    pk1 = _pack(aux1[:, :_H], aux1[:, _H:2 * _H]).reshape(-1)
    msrc_s, mdst_s, mee_s, cnts = _sc_match(pk1, src, dst)
    out1 = _sc_agg1(msrc_s, mdst_s, mee_s, cnts,
                    feat1).reshape(_NP, _W1W)[:_N]
    feat2, aux2 = _tc_b(out1, b1.reshape(1, _W1W), w2r, a2cmb)
    pk2 = _pack(aux2[:, 0:1], aux2[:, 1:2]).reshape(-1)
    out2 = _sc_agg2(msrc_s, mdst_s, cnts, pk2,
                    feat2).reshape(_NP, _C2)[:_N]
    _, probs = _tc_c(out2, b2.reshape(1, _C2), wdp, bdp)
    return probs[:, :_NL]


# speedup vs baseline: 1.3686x; 1.3686x over previous
import functools

import jax
import jax.numpy as jnp
from jax import lax
from jax.experimental import pallas as pl
from jax.experimental.pallas import tpu as pltpu
from jax.experimental.pallas import tpu_sc as plsc

_N = 10000
_E = 320000
_D = 128
_H = 8
_C1 = 64
_C2 = 64
_NL = 40

_NTILES = 32
_EPT = _E // _NTILES
_SUB = 2000
_BE = 1600
_R1 = 157
_NSW1 = 2
_NP1 = _R1 * _NTILES * _NSW1
_R2 = 313
_NP2 = _R2 * _NTILES
_W1W = _H * _C1
_MASKHI = -65536

_GDN = lax.GatherDimensionNumbers(
    offset_dims=(), collapsed_slice_dims=(0,), start_index_map=(0,))


def _lane(v, i):
    idx = jnp.full((16, 1), i, dtype=jnp.int32)
    return lax.gather(v, idx, dimension_numbers=_GDN, slice_sizes=(1,),
                      mode=lax.GatherScatterMode.PROMISE_IN_BOUNDS)


def _mesh():
    return plsc.VectorSubcoreMesh(core_axis_name="c", subcore_axis_name="s",
                                  num_cores=2, num_subcores=16)


def _wid():
    return lax.axis_index("s") * 2 + lax.axis_index("c")


def _ee1_body(pk_h, src_h, dst_h, ee_h, ptab, srcb, dstb, eeb):
    wid = _wid()
    base = wid * _EPT
    iota = lax.iota(jnp.int32, 16)
    pltpu.sync_copy(pk_h, ptab)
    pltpu.sync_copy(src_h.at[pl.ds(base, _EPT)], srcb)
    pltpu.sync_copy(dst_h.at[pl.ds(base, _EPT)], dstb)
    for sb in range(_EPT // _SUB):
        def grp(g, _, sb=sb):
            off = sb * _SUB + g * 16
            s16 = srcb[pl.ds(off, 16)]
            d16 = dstb[pl.ds(off, 16)]
            erel = g * 16 + iota
            for h in range(_H):
                pd = plsc.load_gather(ptab, [d16 * _H + h])
                ps = plsc.load_gather(ptab, [s16 * _H + h])
                a_self = plsc.bitcast(pd & _MASKHI, jnp.float32)
                a_nei = plsc.bitcast(ps << 16, jnp.float32)
                e = a_self + a_nei
                e = jnp.where(e >= 0.0, e, 0.2 * e)
                plsc.store_scatter(eeb, [erel * _H + h], jnp.exp(e))
            return 0
        lax.fori_loop(0, _SUB // 16, grp, 0)
        pltpu.sync_copy(
            eeb, ee_h.at[pl.ds((base + sb * _SUB) * _H, _SUB * _H)])


def _sc_ee1(pk1, src, dst):
    f = pl.kernel(
        _ee1_body,
        out_type=jax.ShapeDtypeStruct((_E * _H,), jnp.float32),
        mesh=_mesh(),
        compiler_params=pltpu.CompilerParams(needs_layout_passes=False),
        scratch_types=[
            pltpu.VMEM((_N * _H,), jnp.int32),
            pltpu.VMEM((_EPT,), jnp.int32),
            pltpu.VMEM((_EPT,), jnp.int32),
            pltpu.VMEM((_SUB * _H,), jnp.float32),
        ],
    )
    return f(pk1, src, dst)


def _ee2_body(pk_h, src_h, dst_h, ee_h, ptab, srcb, dstb, eeb):
    wid = _wid()
    base = wid * _EPT
    pltpu.sync_copy(pk_h, ptab)
    pltpu.sync_copy(src_h.at[pl.ds(base, _EPT)], srcb)
    pltpu.sync_copy(dst_h.at[pl.ds(base, _EPT)], dstb)
    for sb in range(_EPT // _SUB):
        def grp(g, _, sb=sb):
            off = sb * _SUB + g * 16
            s16 = srcb[pl.ds(off, 16)]
            d16 = dstb[pl.ds(off, 16)]
            pd = plsc.load_gather(ptab, [d16])
            ps = plsc.load_gather(ptab, [s16])
            a_self = plsc.bitcast(pd & _MASKHI, jnp.float32)
            a_nei = plsc.bitcast(ps << 16, jnp.float32)
            e = a_self + a_nei
            e = jnp.where(e >= 0.0, e, 0.2 * e)
            eeb[pl.ds(g * 16, 16)] = jnp.exp(e)
            return 0
        lax.fori_loop(0, _SUB // 16, grp, 0)
        pltpu.sync_copy(eeb, ee_h.at[pl.ds(base + sb * _SUB, _SUB)])


def _sc_ee2(pk2, src, dst):
    f = pl.kernel(
        _ee2_body,
        out_type=jax.ShapeDtypeStruct((_E,), jnp.float32),
        mesh=_mesh(),
        compiler_params=pltpu.CompilerParams(needs_layout_passes=False),
        scratch_types=[
            pltpu.VMEM((_N,), jnp.int32),
            pltpu.VMEM((_EPT,), jnp.int32),
            pltpu.VMEM((_EPT,), jnp.int32),
            pltpu.VMEM((_SUB,), jnp.float32),
        ],
    )
    return f(pk2, src, dst)


def _agg1_body(src_h, dst_h, ee_h, feat_h, out_h,
               srcb0, dstb0, eeb0, srcb1, dstb1, eeb1, msrc, mdst, meid,
               fbuf, accum, den, sem, sem2):
    wid = _wid()
    srcbs = (srcb0, srcb1)
    dstbs = (dstb0, dstb1)
    eebs = (eeb0, eeb1)
    iota = lax.iota(jnp.int32, 16)
    zero16 = jnp.zeros((16,), jnp.float32)
    for sweep in range(_NSW1):
        lo = sweep * (_R1 * _NTILES) + wid * _R1

        def zrow(r, _):
            for cc in range(_W1W // 16):
                accum[pl.ds(r * _W1W + cc * 16, 16)] = zero16
            return 0
        lax.fori_loop(0, _R1 + 1, zrow, 0)

        def zden(i, _):
            den[pl.ds(i * 16, 16)] = zero16
            return 0
        lax.fori_loop(0, (_R1 + 1) * _H // 16, zden, 0)

        pltpu.async_copy(src_h.at[pl.ds(0, _BE)], srcbs[0], sem2)
        pltpu.async_copy(dst_h.at[pl.ds(0, _BE)], dstbs[0], sem2)
        pltpu.async_copy(ee_h.at[pl.ds(0, _BE * _H)], eebs[0], sem2)

        def blk2(b2, _, lo=lo):
          for u in range(2):
            b = b2 * 2 + u
            eb = b * _BE
            srcb = srcbs[u]
            dstb = dstbs[u]
            eeblk = eebs[u]
            pltpu.make_async_copy(src_h.at[pl.ds(0, _BE)], srcb, sem2).wait()
            pltpu.make_async_copy(dst_h.at[pl.ds(0, _BE)], dstb, sem2).wait()
            pltpu.make_async_copy(ee_h.at[pl.ds(0, _BE * _H)], eeblk,
                                  sem2).wait()

            @pl.when(b + 1 < _E // _BE)
            def _(eb=eb, u=u):
                pltpu.async_copy(src_h.at[pl.ds(eb + _BE, _BE)],
                                 srcbs[1 - u], sem2)
                pltpu.async_copy(dst_h.at[pl.ds(eb + _BE, _BE)],
                                 dstbs[1 - u], sem2)
                pltpu.async_copy(ee_h.at[pl.ds((eb + _BE) * _H, _BE * _H)],
                                 eebs[1 - u], sem2)

            def grp(g, cur, srcb=srcb, dstb=dstb):
                off = g * 16
                s16 = srcb[pl.ds(off, 16)]
                d16 = dstb[pl.ds(off, 16)]
                dr = d16 - lo
                m = (dr >= 0) & (dr < _R1)
                plsc.store_compressed(msrc.at[pl.ds(cur, 16)], s16, mask=m)
                plsc.store_compressed(mdst.at[pl.ds(cur, 16)], dr, mask=m)
                plsc.store_compressed(meid.at[pl.ds(cur, 16)], off + iota,
                                      mask=m)
                return cur + jnp.max(plsc.all_reduce_population_count(m))
            k = lax.fori_loop(0, _BE // 16, grp, jnp.int32(0))
            msrc[pl.ds(k, 16)] = jnp.zeros((16,), jnp.int32)
            mdst[pl.ds(k, 16)] = jnp.full((16,), _R1, jnp.int32)
            meid[pl.ds(k, 16)] = jnp.zeros((16,), jnp.int32)
            ng = (k + 15) // 16

            def proc(j, _, eeblk=eeblk):
                jo = j * 16
                pltpu.async_copy(feat_h.at[msrc.at[pl.ds(jo, 16)]], fbuf,
                                 sem).wait()
                mei = meid[pl.ds(jo, 16)]
                md = mdst[pl.ds(jo, 16)]

                def edge(g2, _2):
                    er = _lane(mei, g2)
                    db = _lane(md, g2)
                    eerow = plsc.load_gather(eeblk, [er * _H + iota],
                                             mask=iota < _H)
                    plsc.addupdate_scatter(den, [db * _H + iota], eerow,
                                           mask=iota < _H)
                    dbase = db * _W1W
                    for h in range(_H):
                        w = _lane(eerow, h)
                        for cc in range(4):
                            o = h * _C1 + cc * 16
                            v = fbuf[g2, pl.ds(o, 16)]
                            plsc.addupdate_scatter(accum, [dbase + o + iota],
                                                   w * v)
                    return 0
                lax.fori_loop(0, 16, edge, 0)
                return 0
            lax.fori_loop(0, ng, proc, 0)
          return 0
        lax.fori_loop(0, _E // _BE // 2, blk2, 0)

        def drow(r2, _):
            d16 = den[pl.ds(r2 * 16, 16)]
            rec = 1.0 / (d16 + 1e-9)
            for rr in range(2):
                row = r2 * 2 + rr
                for h in range(_H):
                    w = _lane(rec, rr * _H + h)
                    for cc in range(4):
                        o = row * _W1W + h * _C1 + cc * 16
                        accum[pl.ds(o, 16)] = accum[pl.ds(o, 16)] * w
            return 0
        lax.fori_loop(0, (_R1 + 1) // 2, drow, 0)
        pltpu.sync_copy(accum.at[pl.ds(0, _R1 * _W1W)],
                        out_h.at[pl.ds(lo * _W1W, _R1 * _W1W)])


def _sc_agg1(src, dst, ee1, feat1):
    f = pl.kernel(
        _agg1_body,
        out_type=jax.ShapeDtypeStruct((_NP1 * _W1W,), jnp.float32),
        mesh=_mesh(),
        compiler_params=pltpu.CompilerParams(needs_layout_passes=False),
        scratch_types=[
            pltpu.VMEM((_BE,), jnp.int32),
            pltpu.VMEM((_BE,), jnp.int32),
            pltpu.VMEM((_BE * _H,), jnp.float32),
            pltpu.VMEM((_BE,), jnp.int32),
            pltpu.VMEM((_BE,), jnp.int32),
            pltpu.VMEM((_BE * _H,), jnp.float32),
            pltpu.VMEM((_BE + 16,), jnp.int32),
            pltpu.VMEM((_BE + 16,), jnp.int32),
            pltpu.VMEM((_BE + 16,), jnp.int32),
            pltpu.VMEM((16, _W1W), jnp.float32),
            pltpu.VMEM(((_R1 + 1) * _W1W,), jnp.float32),
            pltpu.VMEM(((_R1 + 1) * _H, ), jnp.float32),
            pltpu.SemaphoreType.DMA,
            pltpu.SemaphoreType.DMA,
        ],
    )
    return f(src, dst, ee1, feat1)


def _agg2_body(src_h, dst_h, ee_h, feat_h, out_h,
               srcb0, dstb0, eeb0, srcb1, dstb1, eeb1, msrc, mdst, meid,
               fbuf, accum, den, sem, sem2):
    wid = _wid()
    srcbs = (srcb0, srcb1)
    dstbs = (dstb0, dstb1)
    eebs = (eeb0, eeb1)
    iota = lax.iota(jnp.int32, 16)
    zero16 = jnp.zeros((16,), jnp.float32)
    lo = wid * _R2
    nrow_pad = 320

    def zrow(r, _):
        for cc in range(_C2 // 16):
            accum[pl.ds(r * _C2 + cc * 16, 16)] = zero16
        return 0
    lax.fori_loop(0, nrow_pad, zrow, 0)

    def zden(i, _):
        den[pl.ds(i * 16, 16)] = zero16
        return 0
    lax.fori_loop(0, nrow_pad // 16, zden, 0)

    pltpu.async_copy(src_h.at[pl.ds(0, _BE)], srcbs[0], sem2)
    pltpu.async_copy(dst_h.at[pl.ds(0, _BE)], dstbs[0], sem2)
    pltpu.async_copy(ee_h.at[pl.ds(0, _BE)], eebs[0], sem2)

    def blk2(b2, _):
      for u in range(2):
        b = b2 * 2 + u
        eb = b * _BE
        srcb = srcbs[u]
        dstb = dstbs[u]
        eeblk = eebs[u]
        pltpu.make_async_copy(src_h.at[pl.ds(0, _BE)], srcb, sem2).wait()
        pltpu.make_async_copy(dst_h.at[pl.ds(0, _BE)], dstb, sem2).wait()
        pltpu.make_async_copy(ee_h.at[pl.ds(0, _BE)], eeblk, sem2).wait()

        @pl.when(b + 1 < _E // _BE)
        def _(eb=eb, u=u):
            pltpu.async_copy(src_h.at[pl.ds(eb + _BE, _BE)],
                             srcbs[1 - u], sem2)
            pltpu.async_copy(dst_h.at[pl.ds(eb + _BE, _BE)],
                             dstbs[1 - u], sem2)
            pltpu.async_copy(ee_h.at[pl.ds(eb + _BE, _BE)],
                             eebs[1 - u], sem2)

        def grp(g, cur, srcb=srcb, dstb=dstb):
            off = g * 16
            s16 = srcb[pl.ds(off, 16)]
            d16 = dstb[pl.ds(off, 16)]
            dr = d16 - lo
            m = (dr >= 0) & (dr < _R2)
            plsc.store_compressed(msrc.at[pl.ds(cur, 16)], s16, mask=m)
            plsc.store_compressed(mdst.at[pl.ds(cur, 16)], dr, mask=m)
            plsc.store_compressed(meid.at[pl.ds(cur, 16)], off + iota, mask=m)
            return cur + jnp.max(plsc.all_reduce_population_count(m))
        k = lax.fori_loop(0, _BE // 16, grp, jnp.int32(0))
        msrc[pl.ds(k, 16)] = jnp.zeros((16,), jnp.int32)
        mdst[pl.ds(k, 16)] = jnp.full((16,), _R2, jnp.int32)
        meid[pl.ds(k, 16)] = jnp.zeros((16,), jnp.int32)
        ng = (k + 15) // 16

        def proc(j, _, eeblk=eeblk):
            jo = j * 16
            pltpu.async_copy(feat_h.at[msrc.at[pl.ds(jo, 16)]], fbuf,
                             sem).wait()
            mei = meid[pl.ds(jo, 16)]
            md = mdst[pl.ds(jo, 16)]
            w16 = plsc.load_gather(eeblk, [mei])

            def edge(g2, _2):
                w = _lane(w16, g2)
                db = _lane(md, g2)
                plsc.addupdate_scatter(den, [db], w, mask=iota == 0)
                dbase = db * _C2
                for cc in range(4):
                    o = cc * 16
                    v = fbuf[g2, pl.ds(o, 16)]
                    plsc.addupdate_scatter(accum, [dbase + o + iota], w * v)
                return 0
            lax.fori_loop(0, 16, edge, 0)
            return 0
        lax.fori_loop(0, ng, proc, 0)
      return 0
    lax.fori_loop(0, _E // _BE // 2, blk2, 0)

    def drow(r16, _):
        d16 = den[pl.ds(r16 * 16, 16)]
        rec = 1.0 / (d16 + 1e-9)
        for rr in range(16):
            row = r16 * 16 + rr
            w = _lane(rec, rr)
            for cc in range(4):
                o = row * _C2 + cc * 16
                accum[pl.ds(o, 16)] = accum[pl.ds(o, 16)] * w
        return 0
    lax.fori_loop(0, nrow_pad // 16, drow, 0)
    pltpu.sync_copy(accum.at[pl.ds(0, _R2 * _C2)],
                    out_h.at[pl.ds(lo * _C2, _R2 * _C2)])


def _sc_agg2(src, dst, ee2, feat2):
    f = pl.kernel(
        _agg2_body,
        out_type=jax.ShapeDtypeStruct((_NP2 * _C2,), jnp.float32),
        mesh=_mesh(),
        compiler_params=pltpu.CompilerParams(needs_layout_passes=False),
        scratch_types=[
            pltpu.VMEM((_BE,), jnp.int32),
            pltpu.VMEM((_BE,), jnp.int32),
            pltpu.VMEM((_BE,), jnp.float32),
            pltpu.VMEM((_BE,), jnp.int32),
            pltpu.VMEM((_BE,), jnp.int32),
            pltpu.VMEM((_BE,), jnp.float32),
            pltpu.VMEM((_BE + 16,), jnp.int32),
            pltpu.VMEM((_BE + 16,), jnp.int32),
            pltpu.VMEM((_BE + 16,), jnp.int32),
            pltpu.VMEM((16, 128), jnp.float32),
            pltpu.VMEM((320 * _C2,), jnp.float32),
            pltpu.VMEM((320,), jnp.float32),
            pltpu.SemaphoreType.DMA,
            pltpu.SemaphoreType.DMA,
        ],
    )
    return f(src, dst, ee2, feat2)


_BM = 400


def _tca_body(x_ref, w_ref, a_ref, f_ref, aux_ref):
    f = jnp.dot(x_ref[...], w_ref[...], preferred_element_type=jnp.float32)
    f_ref[...] = f
    aux_ref[...] = jnp.dot(f, a_ref[...], preferred_element_type=jnp.float32)


def _tc_a(x, w1r, acmb):
    return pl.pallas_call(
        _tca_body,
        grid=(_N // _BM,),
        in_specs=[pl.BlockSpec((_BM, _D), lambda i: (i, 0)),
                  pl.BlockSpec((_D, _W1W), lambda i: (0, 0)),
                  pl.BlockSpec((_W1W, 128), lambda i: (0, 0))],
        out_specs=[pl.BlockSpec((_BM, _W1W), lambda i: (i, 0)),
                   pl.BlockSpec((_BM, 128), lambda i: (i, 0))],
        out_shape=[jax.ShapeDtypeStruct((_N, _W1W), jnp.float32),
                   jax.ShapeDtypeStruct((_N, 128), jnp.float32)],
    )(x, w1r, acmb)


def _tcb_body(o1_ref, b1_ref, w2_ref, a2_ref, f2_ref, aux2_ref):
    v = o1_ref[...] + b1_ref[...]
    h1 = jnp.where(v > 0.0, v, jnp.exp(v) - 1.0)
    f2 = jnp.dot(h1, w2_ref[...], preferred_element_type=jnp.float32)
    f2_ref[...] = f2
    aux2_ref[...] = jnp.dot(f2, a2_ref[...],
                            preferred_element_type=jnp.float32)


def _tc_b(o1, b1r, w2r, a2cmb):
    return pl.pallas_call(
        _tcb_body,
        grid=(_N // _BM,),
        in_specs=[pl.BlockSpec((_BM, _W1W), lambda i: (i, 0)),
                  pl.BlockSpec((1, _W1W), lambda i: (0, 0)),
                  pl.BlockSpec((_W1W, 128), lambda i: (0, 0)),
                  pl.BlockSpec((128, 128), lambda i: (0, 0))],
        out_specs=[pl.BlockSpec((_BM, 128), lambda i: (i, 0)),
                   pl.BlockSpec((_BM, 128), lambda i: (i, 0))],
        out_shape=[jax.ShapeDtypeStruct((_N, 128), jnp.float32),
                   jax.ShapeDtypeStruct((_N, 128), jnp.float32)],
    )(o1, b1r, w2r, a2cmb)


def _tcc_body(o2_ref, b2_ref, wd_ref, bd_ref, g_ref, l_ref):
    i = pl.program_id(0)
    v = o2_ref[...] + b2_ref[...]
    h2 = jnp.where(v > 0.0, v, jnp.exp(v) - 1.0)
    ps = jnp.sum(h2, axis=0, keepdims=True)

    @pl.when(i == 0)
    def _():
        g_ref[...] = ps

    @pl.when(i > 0)
    def _():
        g_ref[...] = g_ref[...] + ps

    @pl.when(i == pl.num_programs(0) - 1)
    def _():
        l = jnp.dot(g_ref[...], wd_ref[...],
                    preferred_element_type=jnp.float32) + bd_ref[...]
        m = jnp.max(l, axis=-1, keepdims=True)
        z = jnp.exp(l - m)
        l_ref[...] = z / jnp.sum(z, axis=-1, keepdims=True)


def _tc_c(o2, b2r, wdp, bdp):
    bm = 400
    return pl.pallas_call(
        _tcc_body,
        grid=(_N // bm,),
        in_specs=[pl.BlockSpec((bm, _C2), lambda i: (i, 0)),
                  pl.BlockSpec((1, _C2), lambda i: (0, 0)),
                  pl.BlockSpec((_C2, 128), lambda i: (0, 0)),
                  pl.BlockSpec((1, 128), lambda i: (0, 0))],
        out_specs=[pl.BlockSpec((1, _C2), lambda i: (0, 0)),
                   pl.BlockSpec((1, 128), lambda i: (0, 0))],
        out_shape=[jax.ShapeDtypeStruct((1, _C2), jnp.float32),
                   jax.ShapeDtypeStruct((1, 128), jnp.float32)],
    )(o2, b2r, wdp, bdp)


def _pack(a_hi, a_lo):
    hi = lax.bitcast_convert_type(a_hi.astype(jnp.bfloat16),
                                  jnp.uint16).astype(jnp.uint32) << 16
    lo = lax.bitcast_convert_type(a_lo.astype(jnp.bfloat16),
                                  jnp.uint16).astype(jnp.uint32)
    return lax.bitcast_convert_type(hi | lo, jnp.int32)


def kernel(x, edge_index, W1, a_src1, a_dst1, b1, W2, a_src2, a_dst2, b2,
           Wd, bd):
    src = edge_index[0]
    dst = edge_index[1]
    w1r = W1.reshape(_D, _W1W)
    eye8 = jnp.eye(_H, dtype=jnp.float32)
    acmb = jnp.concatenate([
        jnp.einsum("hc,hk->hck", a_src1, eye8).reshape(_W1W, _H),
        jnp.einsum("hc,hk->hck", a_dst1, eye8).reshape(_W1W, _H),
    ], axis=1)
    acmb = jnp.pad(acmb, ((0, 0), (0, 128 - 2 * _H)))
    w2r = jnp.pad(W2.reshape(_W1W, _C2), ((0, 0), (0, 128 - _C2)))
    a2cmb = jnp.pad(jnp.concatenate([a_src2.T, a_dst2.T], axis=1),
                    ((0, 128 - _C2), (0, 126)))
    wdp = jnp.pad(Wd, ((0, 0), (0, 128 - _NL)))
    bdp = jnp.concatenate(
        [bd, jnp.full((128 - _NL,), -1e30, jnp.float32)]).reshape(1, 128)

    feat1, aux1 = _tc_a(x, w1r, acmb)
    pk1 = _pack(aux1[:, :_H], aux1[:, _H:2 * _H]).reshape(-1)
    ee1 = _sc_ee1(pk1, src, dst)
    out1 = _sc_agg1(src, dst, ee1, feat1).reshape(_NP1, _W1W)[:_N]
    feat2, aux2 = _tc_b(out1, b1.reshape(1, _W1W), w2r, a2cmb)
    pk2 = _pack(aux2[:, 0:1], aux2[:, 1:2]).reshape(-1)
    ee2 = _sc_ee2(pk2, src, dst)
    out2 = _sc_agg2(src, dst, ee2, feat2).reshape(_NP2, _C2)[:_N]
    _, probs = _tc_c(out2, b2.reshape(1, _C2), wdp, bdp)
    return probs[:, :_NL]


# speedup vs baseline: 1.6315x; 1.1920x over previous
import functools

import jax
import jax.numpy as jnp
from jax import lax
from jax.experimental import pallas as pl
from jax.experimental.pallas import tpu as pltpu
from jax.experimental.pallas import tpu_sc as plsc

_N = 10000
_E = 320000
_D = 128
_H = 8
_C1 = 64
_C2 = 64
_NL = 40

_NTILES = 32
_EPT = _E // _NTILES
_SUB = 2000
_BE = 2000
_R1 = 157
_NSW1 = 2
_NP1 = _R1 * _NTILES * _NSW1
_R2 = 313
_NP2 = _R2 * _NTILES
_W1W = _H * _C1
_MASKHI = -65536

_GDN = lax.GatherDimensionNumbers(
    offset_dims=(), collapsed_slice_dims=(0,), start_index_map=(0,))


def _lane(v, i):
    idx = jnp.full((16, 1), i, dtype=jnp.int32)
    return lax.gather(v, idx, dimension_numbers=_GDN, slice_sizes=(1,),
                      mode=lax.GatherScatterMode.PROMISE_IN_BOUNDS)


def _mesh():
    return plsc.VectorSubcoreMesh(core_axis_name="c", subcore_axis_name="s",
                                  num_cores=2, num_subcores=16)


def _wid():
    return lax.axis_index("s") * 2 + lax.axis_index("c")


def _ee1_body(pk_h, src_h, dst_h, ee_h, ptab, srcb, dstb, eeb):
    wid = _wid()
    base = wid * _EPT
    iota = lax.iota(jnp.int32, 16)
    pltpu.sync_copy(pk_h, ptab)
    pltpu.sync_copy(src_h.at[pl.ds(base, _EPT)], srcb)
    pltpu.sync_copy(dst_h.at[pl.ds(base, _EPT)], dstb)
    for sb in range(_EPT // _SUB):
        def grp(g, _, sb=sb):
            off = sb * _SUB + g * 16
            s16 = srcb[pl.ds(off, 16)]
            d16 = dstb[pl.ds(off, 16)]
            erel = g * 16 + iota
            for h in range(_H):
                pd = plsc.load_gather(ptab, [d16 * _H + h])
                ps = plsc.load_gather(ptab, [s16 * _H + h])
                a_self = plsc.bitcast(pd & _MASKHI, jnp.float32)
                a_nei = plsc.bitcast(ps << 16, jnp.float32)
                e = a_self + a_nei
                e = jnp.where(e >= 0.0, e, 0.2 * e)
                plsc.store_scatter(eeb, [erel * _H + h], jnp.exp(e))
            return 0
        lax.fori_loop(0, _SUB // 16, grp, 0)
        pltpu.sync_copy(
            eeb, ee_h.at[pl.ds((base + sb * _SUB) * _H, _SUB * _H)])


def _sc_ee1(pk1, src, dst):
    f = pl.kernel(
        _ee1_body,
        out_type=jax.ShapeDtypeStruct((_E * _H,), jnp.float32),
        mesh=_mesh(),
        compiler_params=pltpu.CompilerParams(needs_layout_passes=False),
        scratch_types=[
            pltpu.VMEM((_N * _H,), jnp.int32),
            pltpu.VMEM((_EPT,), jnp.int32),
            pltpu.VMEM((_EPT,), jnp.int32),
            pltpu.VMEM((_SUB * _H,), jnp.float32),
        ],
    )
    return f(pk1, src, dst)


def _ee2_body(pk_h, src_h, dst_h, ee_h, ptab, srcb, dstb, eeb):
    wid = _wid()
    base = wid * _EPT
    pltpu.sync_copy(pk_h, ptab)
    pltpu.sync_copy(src_h.at[pl.ds(base, _EPT)], srcb)
    pltpu.sync_copy(dst_h.at[pl.ds(base, _EPT)], dstb)
    for sb in range(_EPT // _SUB):
        def grp(g, _, sb=sb):
            off = sb * _SUB + g * 16
            s16 = srcb[pl.ds(off, 16)]
            d16 = dstb[pl.ds(off, 16)]
            pd = plsc.load_gather(ptab, [d16])
            ps = plsc.load_gather(ptab, [s16])
            a_self = plsc.bitcast(pd & _MASKHI, jnp.float32)
            a_nei = plsc.bitcast(ps << 16, jnp.float32)
            e = a_self + a_nei
            e = jnp.where(e >= 0.0, e, 0.2 * e)
            eeb[pl.ds(g * 16, 16)] = jnp.exp(e)
            return 0
        lax.fori_loop(0, _SUB // 16, grp, 0)
        pltpu.sync_copy(eeb, ee_h.at[pl.ds(base + sb * _SUB, _SUB)])


def _sc_ee2(pk2, src, dst):
    f = pl.kernel(
        _ee2_body,
        out_type=jax.ShapeDtypeStruct((_E,), jnp.float32),
        mesh=_mesh(),
        compiler_params=pltpu.CompilerParams(needs_layout_passes=False),
        scratch_types=[
            pltpu.VMEM((_N,), jnp.int32),
            pltpu.VMEM((_EPT,), jnp.int32),
            pltpu.VMEM((_EPT,), jnp.int32),
            pltpu.VMEM((_SUB,), jnp.float32),
        ],
    )
    return f(pk2, src, dst)


def _agg1_body(src_h, dst_h, ee_h, feat_h, out_h,
               srcb, dstb, eeblk, msrc, mdst, meid, fbuf, accum, den, sem):
    wid = _wid()
    iota = lax.iota(jnp.int32, 16)
    zero16 = jnp.zeros((16,), jnp.float32)
    for sweep in range(_NSW1):
        lo = sweep * (_R1 * _NTILES) + wid * _R1

        def zrow(r, _):
            for cc in range(_W1W // 16):
                accum[pl.ds(r * _W1W + cc * 16, 16)] = zero16
            return 0
        lax.fori_loop(0, _R1 + 1, zrow, 0)

        def zden(i, _):
            den[pl.ds(i * 16, 16)] = zero16
            return 0
        lax.fori_loop(0, (_R1 + 1) * _H // 16, zden, 0)

        def blk(b, _, lo=lo):
            eb = b * _BE
            pltpu.sync_copy(src_h.at[pl.ds(eb, _BE)], srcb)
            pltpu.sync_copy(dst_h.at[pl.ds(eb, _BE)], dstb)
            pltpu.sync_copy(ee_h.at[pl.ds(eb * _H, _BE * _H)], eeblk)

            def grp(g, cur):
                off = g * 16
                s16 = srcb[pl.ds(off, 16)]
                d16 = dstb[pl.ds(off, 16)]
                dr = d16 - lo
                m = (dr >= 0) & (dr < _R1)
                plsc.store_compressed(msrc.at[pl.ds(cur, 16)], s16, mask=m)
                plsc.store_compressed(mdst.at[pl.ds(cur, 16)], dr, mask=m)
                plsc.store_compressed(meid.at[pl.ds(cur, 16)], off + iota,
                                      mask=m)
                return cur + jnp.max(plsc.all_reduce_population_count(m))
            k = lax.fori_loop(0, _BE // 16, grp, jnp.int32(0))
            msrc[pl.ds(k, 16)] = jnp.zeros((16,), jnp.int32)
            mdst[pl.ds(k, 16)] = jnp.full((16,), _R1, jnp.int32)
            meid[pl.ds(k, 16)] = jnp.zeros((16,), jnp.int32)
            ng = (k + 15) // 16

            def proc(j, _):
                jo = j * 16
                pltpu.async_copy(feat_h.at[msrc.at[pl.ds(jo, 16)]], fbuf,
                                 sem).wait()
                mei = meid[pl.ds(jo, 16)]
                md = mdst[pl.ds(jo, 16)]

                def edge(g2, _2):
                    er = _lane(mei, g2)
                    db = _lane(md, g2)
                    eerow = plsc.load_gather(eeblk, [er * _H + iota],
                                             mask=iota < _H)
                    plsc.addupdate_scatter(den, [db * _H + iota], eerow,
                                           mask=iota < _H)
                    dbase = db * _W1W
                    for h in range(_H):
                        w = _lane(eerow, h)
                        for cc in range(4):
                            o = h * _C1 + cc * 16
                            v = fbuf[g2, pl.ds(o, 16)]
                            plsc.addupdate_scatter(accum, [dbase + o + iota],
                                                   w * v)
                    return 0
                lax.fori_loop(0, 16, edge, 0)
                return 0
            lax.fori_loop(0, ng, proc, 0)
            return 0
        lax.fori_loop(0, _E // _BE, blk, 0)

        def drow(r2, _):
            d16 = den[pl.ds(r2 * 16, 16)]
            rec = 1.0 / (d16 + 1e-9)
            for rr in range(2):
                row = r2 * 2 + rr
                for h in range(_H):
                    w = _lane(rec, rr * _H + h)
                    for cc in range(4):
                        o = row * _W1W + h * _C1 + cc * 16
                        accum[pl.ds(o, 16)] = accum[pl.ds(o, 16)] * w
            return 0
        lax.fori_loop(0, (_R1 + 1) // 2, drow, 0)
        pltpu.sync_copy(accum.at[pl.ds(0, _R1 * _W1W)],
                        out_h.at[pl.ds(lo * _W1W, _R1 * _W1W)])


def _sc_agg1(src, dst, ee1, feat1):
    f = pl.kernel(
        _agg1_body,
        out_type=jax.ShapeDtypeStruct((_NP1 * _W1W,), jnp.float32),
        mesh=_mesh(),
        compiler_params=pltpu.CompilerParams(needs_layout_passes=False),
        scratch_types=[
            pltpu.VMEM((_BE,), jnp.int32),
            pltpu.VMEM((_BE,), jnp.int32),
            pltpu.VMEM((_BE * _H,), jnp.float32),
            pltpu.VMEM((_BE + 16,), jnp.int32),
            pltpu.VMEM((_BE + 16,), jnp.int32),
            pltpu.VMEM((_BE + 16,), jnp.int32),
            pltpu.VMEM((16, _W1W), jnp.float32),
            pltpu.VMEM(((_R1 + 1) * _W1W,), jnp.float32),
            pltpu.VMEM(((_R1 + 1) * _H, ), jnp.float32),
            pltpu.SemaphoreType.DMA,
        ],
    )
    return f(src, dst, ee1, feat1)


_BE2 = 4000


def _agg2_body(src_h, dst_h, ee_h, feat_h, out_h,
               srcb0, dstb0, eeb0, srcb1, dstb1, eeb1, msrc, mdst, meid,
               fbuf, accum, den, sem, sem2):
    wid = _wid()
    srcbs = (srcb0, srcb1)
    dstbs = (dstb0, dstb1)
    eebs = (eeb0, eeb1)
    iota = lax.iota(jnp.int32, 16)
    zero16 = jnp.zeros((16,), jnp.float32)
    lo = wid * _R2
    nrow_pad = 320

    def zrow(r, _):
        for cc in range(_C2 // 16):
            accum[pl.ds(r * _C2 + cc * 16, 16)] = zero16
        return 0
    lax.fori_loop(0, nrow_pad, zrow, 0)

    def zden(i, _):
        den[pl.ds(i * 16, 16)] = zero16
        return 0
    lax.fori_loop(0, nrow_pad // 16, zden, 0)

    pltpu.async_copy(src_h.at[pl.ds(0, _BE2)], srcbs[0], sem2)
    pltpu.async_copy(dst_h.at[pl.ds(0, _BE2)], dstbs[0], sem2)
    pltpu.async_copy(ee_h.at[pl.ds(0, _BE2)], eebs[0], sem2)

    def blk2(b2, _):
      for u in range(2):
        b = b2 * 2 + u
        eb = b * _BE2
        srcb = srcbs[u]
        dstb = dstbs[u]
        eeblk = eebs[u]
        pltpu.make_async_copy(src_h.at[pl.ds(0, _BE2)], srcb, sem2).wait()
        pltpu.make_async_copy(dst_h.at[pl.ds(0, _BE2)], dstb, sem2).wait()
        pltpu.make_async_copy(ee_h.at[pl.ds(0, _BE2)], eeblk, sem2).wait()

        @pl.when(b + 1 < _E // _BE2)
        def _(eb=eb, u=u):
            pltpu.async_copy(src_h.at[pl.ds(eb + _BE2, _BE2)],
                             srcbs[1 - u], sem2)
            pltpu.async_copy(dst_h.at[pl.ds(eb + _BE2, _BE2)],
                             dstbs[1 - u], sem2)
            pltpu.async_copy(ee_h.at[pl.ds(eb + _BE2, _BE2)],
                             eebs[1 - u], sem2)

        def grp(g, cur, srcb=srcb, dstb=dstb):
            off = g * 16
            s16 = srcb[pl.ds(off, 16)]
            d16 = dstb[pl.ds(off, 16)]
            dr = d16 - lo
            m = (dr >= 0) & (dr < _R2)
            plsc.store_compressed(msrc.at[pl.ds(cur, 16)], s16, mask=m)
            plsc.store_compressed(mdst.at[pl.ds(cur, 16)], dr, mask=m)
            plsc.store_compressed(meid.at[pl.ds(cur, 16)], off + iota, mask=m)
            return cur + jnp.max(plsc.all_reduce_population_count(m))
        k = lax.fori_loop(0, _BE2 // 16, grp, jnp.int32(0))
        msrc[pl.ds(k, 16)] = jnp.zeros((16,), jnp.int32)
        mdst[pl.ds(k, 16)] = jnp.full((16,), _R2, jnp.int32)
        meid[pl.ds(k, 16)] = jnp.zeros((16,), jnp.int32)
        ng = (k + 15) // 16

        def proc(j, _, eeblk=eeblk):
            jo = j * 16
            pltpu.async_copy(feat_h.at[msrc.at[pl.ds(jo, 16)]], fbuf,
                             sem).wait()
            mei = meid[pl.ds(jo, 16)]
            md = mdst[pl.ds(jo, 16)]
            w16 = plsc.load_gather(eeblk, [mei])

            def edge(g2, _2):
                w = _lane(w16, g2)
                db = _lane(md, g2)
                plsc.addupdate_scatter(den, [db], w, mask=iota == 0)
                dbase = db * _C2
                for cc in range(4):
                    o = cc * 16
                    v = fbuf[g2, pl.ds(o, 16)]
                    plsc.addupdate_scatter(accum, [dbase + o + iota], w * v)
                return 0
            lax.fori_loop(0, 16, edge, 0)
            return 0
        lax.fori_loop(0, ng, proc, 0)
      return 0
    lax.fori_loop(0, _E // _BE2 // 2, blk2, 0)

    def drow(r16, _):
        d16 = den[pl.ds(r16 * 16, 16)]
        rec = 1.0 / (d16 + 1e-9)
        for rr in range(16):
            row = r16 * 16 + rr
            w = _lane(rec, rr)
            for cc in range(4):
                o = row * _C2 + cc * 16
                accum[pl.ds(o, 16)] = accum[pl.ds(o, 16)] * w
        return 0
    lax.fori_loop(0, nrow_pad // 16, drow, 0)
    pltpu.sync_copy(accum.at[pl.ds(0, _R2 * _C2)],
                    out_h.at[pl.ds(lo * _C2, _R2 * _C2)])


def _sc_agg2(src, dst, ee2, feat2):
    f = pl.kernel(
        _agg2_body,
        out_type=jax.ShapeDtypeStruct((_NP2 * _C2,), jnp.float32),
        mesh=_mesh(),
        compiler_params=pltpu.CompilerParams(needs_layout_passes=False),
        scratch_types=[
            pltpu.VMEM((_BE2,), jnp.int32),
            pltpu.VMEM((_BE2,), jnp.int32),
            pltpu.VMEM((_BE2,), jnp.float32),
            pltpu.VMEM((_BE2,), jnp.int32),
            pltpu.VMEM((_BE2,), jnp.int32),
            pltpu.VMEM((_BE2,), jnp.float32),
            pltpu.VMEM((_BE2 + 16,), jnp.int32),
            pltpu.VMEM((_BE2 + 16,), jnp.int32),
            pltpu.VMEM((_BE2 + 16,), jnp.int32),
            pltpu.VMEM((16, 128), jnp.float32),
            pltpu.VMEM((320 * _C2,), jnp.float32),
            pltpu.VMEM((320,), jnp.float32),
            pltpu.SemaphoreType.DMA,
            pltpu.SemaphoreType.DMA,
        ],
    )
    return f(src, dst, ee2, feat2)


_BM = 400


def _tca_body(x_ref, w_ref, a_ref, f_ref, aux_ref):
    f = jnp.dot(x_ref[...], w_ref[...], preferred_element_type=jnp.float32)
    f_ref[...] = f
    aux_ref[...] = jnp.dot(f, a_ref[...], preferred_element_type=jnp.float32)


def _tc_a(x, w1r, acmb):
    return pl.pallas_call(
        _tca_body,
        grid=(_N // _BM,),
        in_specs=[pl.BlockSpec((_BM, _D), lambda i: (i, 0)),
                  pl.BlockSpec((_D, _W1W), lambda i: (0, 0)),
                  pl.BlockSpec((_W1W, 128), lambda i: (0, 0))],
        out_specs=[pl.BlockSpec((_BM, _W1W), lambda i: (i, 0)),
                   pl.BlockSpec((_BM, 128), lambda i: (i, 0))],
        out_shape=[jax.ShapeDtypeStruct((_N, _W1W), jnp.float32),
                   jax.ShapeDtypeStruct((_N, 128), jnp.float32)],
    )(x, w1r, acmb)


def _tcb_body(o1_ref, b1_ref, w2_ref, a2_ref, f2_ref, aux2_ref):
    v = o1_ref[...] + b1_ref[...]
    h1 = jnp.where(v > 0.0, v, jnp.exp(v) - 1.0)
    f2 = jnp.dot(h1, w2_ref[...], preferred_element_type=jnp.float32)
    f2_ref[...] = f2
    aux2_ref[...] = jnp.dot(f2, a2_ref[...],
                            preferred_element_type=jnp.float32)


def _tc_b(o1, b1r, w2r, a2cmb):
    return pl.pallas_call(
        _tcb_body,
        grid=(_N // _BM,),
        in_specs=[pl.BlockSpec((_BM, _W1W), lambda i: (i, 0)),
                  pl.BlockSpec((1, _W1W), lambda i: (0, 0)),
                  pl.BlockSpec((_W1W, 128), lambda i: (0, 0)),
                  pl.BlockSpec((128, 128), lambda i: (0, 0))],
        out_specs=[pl.BlockSpec((_BM, 128), lambda i: (i, 0)),
                   pl.BlockSpec((_BM, 128), lambda i: (i, 0))],
        out_shape=[jax.ShapeDtypeStruct((_N, 128), jnp.float32),
                   jax.ShapeDtypeStruct((_N, 128), jnp.float32)],
    )(o1, b1r, w2r, a2cmb)


def _tcc_body(o2_ref, b2_ref, wd_ref, bd_ref, g_ref, l_ref):
    i = pl.program_id(0)
    v = o2_ref[...] + b2_ref[...]
    h2 = jnp.where(v > 0.0, v, jnp.exp(v) - 1.0)
    ps = jnp.sum(h2, axis=0, keepdims=True)

    @pl.when(i == 0)
    def _():
        g_ref[...] = ps

    @pl.when(i > 0)
    def _():
        g_ref[...] = g_ref[...] + ps

    @pl.when(i == pl.num_programs(0) - 1)
    def _():
        l = jnp.dot(g_ref[...], wd_ref[...],
                    preferred_element_type=jnp.float32) + bd_ref[...]
        m = jnp.max(l, axis=-1, keepdims=True)
        z = jnp.exp(l - m)
        l_ref[...] = z / jnp.sum(z, axis=-1, keepdims=True)


def _tc_c(o2, b2r, wdp, bdp):
    bm = 400
    return pl.pallas_call(
        _tcc_body,
        grid=(_N // bm,),
        in_specs=[pl.BlockSpec((bm, _C2), lambda i: (i, 0)),
                  pl.BlockSpec((1, _C2), lambda i: (0, 0)),
                  pl.BlockSpec((_C2, 128), lambda i: (0, 0)),
                  pl.BlockSpec((1, 128), lambda i: (0, 0))],
        out_specs=[pl.BlockSpec((1, _C2), lambda i: (0, 0)),
                   pl.BlockSpec((1, 128), lambda i: (0, 0))],
        out_shape=[jax.ShapeDtypeStruct((1, _C2), jnp.float32),
                   jax.ShapeDtypeStruct((1, 128), jnp.float32)],
    )(o2, b2r, wdp, bdp)


def _pack(a_hi, a_lo):
    hi = lax.bitcast_convert_type(a_hi.astype(jnp.bfloat16),
                                  jnp.uint16).astype(jnp.uint32) << 16
    lo = lax.bitcast_convert_type(a_lo.astype(jnp.bfloat16),
                                  jnp.uint16).astype(jnp.uint32)
    return lax.bitcast_convert_type(hi | lo, jnp.int32)


def kernel(x, edge_index, W1, a_src1, a_dst1, b1, W2, a_src2, a_dst2, b2,
           Wd, bd):
    src = edge_index[0]
    dst = edge_index[1]
    w1r = W1.reshape(_D, _W1W)
    eye8 = jnp.eye(_H, dtype=jnp.float32)
    acmb = jnp.concatenate([
        jnp.einsum("hc,hk->hck", a_src1, eye8).reshape(_W1W, _H),
        jnp.einsum("hc,hk->hck", a_dst1, eye8).reshape(_W1W, _H),
    ], axis=1)
    acmb = jnp.pad(acmb, ((0, 0), (0, 128 - 2 * _H)))
    w2r = jnp.pad(W2.reshape(_W1W, _C2), ((0, 0), (0, 128 - _C2)))
    a2cmb = jnp.pad(jnp.concatenate([a_src2.T, a_dst2.T], axis=1),
                    ((0, 128 - _C2), (0, 126)))
    wdp = jnp.pad(Wd, ((0, 0), (0, 128 - _NL)))
    bdp = jnp.concatenate(
        [bd, jnp.full((128 - _NL,), -1e30, jnp.float32)]).reshape(1, 128)

    feat1, aux1 = _tc_a(x, w1r, acmb)
    pk1 = _pack(aux1[:, :_H], aux1[:, _H:2 * _H]).reshape(-1)
    ee1 = _sc_ee1(pk1, src, dst)
    out1 = _sc_agg1(src, dst, ee1, feat1).reshape(_NP1, _W1W)[:_N]
    feat2, aux2 = _tc_b(out1, b1.reshape(1, _W1W), w2r, a2cmb)
    pk2 = _pack(aux2[:, 0:1], aux2[:, 1:2]).reshape(-1)
    ee2 = _sc_ee2(pk2, src, dst)
    out2 = _sc_agg2(src, dst, ee2, feat2).reshape(_NP2, _C2)[:_N]
    _, probs = _tc_c(out2, b2.reshape(1, _C2), wdp, bdp)
    return probs[:, :_NL]


# speedup vs baseline: 1.7027x; 1.0437x over previous
import functools

import jax
import jax.numpy as jnp
from jax import lax
from jax.experimental import pallas as pl
from jax.experimental.pallas import tpu as pltpu
from jax.experimental.pallas import tpu_sc as plsc

_N = 10000
_E = 320000
_D = 128
_H = 8
_C1 = 64
_C2 = 64
_NL = 40

_NTILES = 32
_EPT = _E // _NTILES
_SUB = 2000
_BE = 2000
_R1 = 157
_NSW1 = 2
_NP1 = _R1 * _NTILES * _NSW1
_R2 = 313
_NP2 = _R2 * _NTILES
_W1W = _H * _C1
_MASKHI = -65536

_GDN = lax.GatherDimensionNumbers(
    offset_dims=(), collapsed_slice_dims=(0,), start_index_map=(0,))


def _lane(v, i):
    idx = jnp.full((16, 1), i, dtype=jnp.int32)
    return lax.gather(v, idx, dimension_numbers=_GDN, slice_sizes=(1,),
                      mode=lax.GatherScatterMode.PROMISE_IN_BOUNDS)


def _mesh():
    return plsc.VectorSubcoreMesh(core_axis_name="c", subcore_axis_name="s",
                                  num_cores=2, num_subcores=16)


def _wid():
    return lax.axis_index("s") * 2 + lax.axis_index("c")


def _ee1_body(pk_h, src_h, dst_h, ee_h, ptab, srcb, dstb, eeb):
    wid = _wid()
    base = wid * _EPT
    iota = lax.iota(jnp.int32, 16)
    pltpu.sync_copy(pk_h, ptab)
    pltpu.sync_copy(src_h.at[pl.ds(base, _EPT)], srcb)
    pltpu.sync_copy(dst_h.at[pl.ds(base, _EPT)], dstb)
    for sb in range(_EPT // _SUB):
        def grp(g, _, sb=sb):
            off = sb * _SUB + g * 16
            s16 = srcb[pl.ds(off, 16)]
            d16 = dstb[pl.ds(off, 16)]
            erel = g * 16 + iota
            for h in range(_H):
                pd = plsc.load_gather(ptab, [d16 * _H + h])
                ps = plsc.load_gather(ptab, [s16 * _H + h])
                a_self = plsc.bitcast(pd & _MASKHI, jnp.float32)
                a_nei = plsc.bitcast(ps << 16, jnp.float32)
                e = a_self + a_nei
                e = jnp.where(e >= 0.0, e, 0.2 * e)
                plsc.store_scatter(eeb, [erel * _H + h], jnp.exp(e))
            return 0
        lax.fori_loop(0, _SUB // 16, grp, 0)
        pltpu.sync_copy(
            eeb, ee_h.at[pl.ds((base + sb * _SUB) * _H, _SUB * _H)])


def _sc_ee1(pk1, src, dst):
    f = pl.kernel(
        _ee1_body,
        out_type=jax.ShapeDtypeStruct((_E * _H,), jnp.float32),
        mesh=_mesh(),
        compiler_params=pltpu.CompilerParams(needs_layout_passes=False),
        scratch_types=[
            pltpu.VMEM((_N * _H,), jnp.int32),
            pltpu.VMEM((_EPT,), jnp.int32),
            pltpu.VMEM((_EPT,), jnp.int32),
            pltpu.VMEM((_SUB * _H,), jnp.float32),
        ],
    )
    return f(pk1, src, dst)


def _ee2_body(pk_h, src_h, dst_h, ee_h, ptab, srcb, dstb, eeb):
    wid = _wid()
    base = wid * _EPT
    pltpu.sync_copy(pk_h, ptab)
    pltpu.sync_copy(src_h.at[pl.ds(base, _EPT)], srcb)
    pltpu.sync_copy(dst_h.at[pl.ds(base, _EPT)], dstb)
    for sb in range(_EPT // _SUB):
        def grp(g, _, sb=sb):
            off = sb * _SUB + g * 16
            s16 = srcb[pl.ds(off, 16)]
            d16 = dstb[pl.ds(off, 16)]
            pd = plsc.load_gather(ptab, [d16])
            ps = plsc.load_gather(ptab, [s16])
            a_self = plsc.bitcast(pd & _MASKHI, jnp.float32)
            a_nei = plsc.bitcast(ps << 16, jnp.float32)
            e = a_self + a_nei
            e = jnp.where(e >= 0.0, e, 0.2 * e)
            eeb[pl.ds(g * 16, 16)] = jnp.exp(e)
            return 0
        lax.fori_loop(0, _SUB // 16, grp, 0)
        pltpu.sync_copy(eeb, ee_h.at[pl.ds(base + sb * _SUB, _SUB)])


def _sc_ee2(pk2, src, dst):
    f = pl.kernel(
        _ee2_body,
        out_type=jax.ShapeDtypeStruct((_E,), jnp.float32),
        mesh=_mesh(),
        compiler_params=pltpu.CompilerParams(needs_layout_passes=False),
        scratch_types=[
            pltpu.VMEM((_N,), jnp.int32),
            pltpu.VMEM((_EPT,), jnp.int32),
            pltpu.VMEM((_EPT,), jnp.int32),
            pltpu.VMEM((_SUB,), jnp.float32),
        ],
    )
    return f(pk2, src, dst)


def _agg1_body(src_h, dst_h, ee_h, feat_h, out_h,
               srcb, dstb, eeblk, msrc, mdst, meid, fbuf, accum, den, sem):
    wid = _wid()
    iota = lax.iota(jnp.int32, 16)
    zero16 = jnp.zeros((16,), jnp.float32)
    for sweep in range(_NSW1):
        lo = sweep * (_R1 * _NTILES) + wid * _R1

        def zrow(r, _):
            for cc in range(_W1W // 16):
                accum[pl.ds(r * _W1W + cc * 16, 16)] = zero16
            return 0
        lax.fori_loop(0, _R1 + 1, zrow, 0)

        def zden(i, _):
            den[pl.ds(i * 16, 16)] = zero16
            return 0
        lax.fori_loop(0, (_R1 + 1) * _H // 16, zden, 0)

        def blk(b, _, lo=lo):
            eb = b * _BE
            pltpu.sync_copy(src_h.at[pl.ds(eb, _BE)], srcb)
            pltpu.sync_copy(dst_h.at[pl.ds(eb, _BE)], dstb)
            pltpu.sync_copy(ee_h.at[pl.ds(eb * _H, _BE * _H)], eeblk)

            def grp(g, cur):
                off = g * 16
                s16 = srcb[pl.ds(off, 16)]
                d16 = dstb[pl.ds(off, 16)]
                dr = d16 - lo
                m = (dr >= 0) & (dr < _R1)
                plsc.store_compressed(msrc.at[pl.ds(cur, 16)], s16, mask=m)
                plsc.store_compressed(mdst.at[pl.ds(cur, 16)], dr, mask=m)
                plsc.store_compressed(meid.at[pl.ds(cur, 16)], off + iota,
                                      mask=m)
                return cur + jnp.max(plsc.all_reduce_population_count(m))
            k = lax.fori_loop(0, _BE // 16, grp, jnp.int32(0))
            msrc[pl.ds(k, 16)] = jnp.zeros((16,), jnp.int32)
            mdst[pl.ds(k, 16)] = jnp.full((16,), _R1, jnp.int32)
            meid[pl.ds(k, 16)] = jnp.zeros((16,), jnp.int32)
            ng = (k + 15) // 16

            def proc(j, _):
                jo = j * 16
                ne = jnp.minimum(16, k - jo)
                pltpu.async_copy(feat_h.at[msrc.at[pl.ds(jo, 16)]], fbuf,
                                 sem).wait()
                mei = meid[pl.ds(jo, 16)]
                md = mdst[pl.ds(jo, 16)]

                def edge(g2, _2):
                    er = _lane(mei, g2)
                    db = _lane(md, g2)
                    eerow = plsc.load_gather(eeblk, [er * _H + iota],
                                             mask=iota < _H)
                    plsc.addupdate_scatter(den, [db * _H + iota], eerow,
                                           mask=iota < _H)
                    dbase = db * _W1W
                    for h in range(_H):
                        w = _lane(eerow, h)
                        for cc in range(4):
                            o = h * _C1 + cc * 16
                            v = fbuf[g2, pl.ds(o, 16)]
                            plsc.addupdate_scatter(accum, [dbase + o + iota],
                                                   w * v)
                    return 0
                lax.fori_loop(0, ne, edge, 0)
                return 0
            lax.fori_loop(0, ng, proc, 0)
            return 0
        lax.fori_loop(0, _E // _BE, blk, 0)

        def drow(r2, _):
            d16 = den[pl.ds(r2 * 16, 16)]
            rec = 1.0 / (d16 + 1e-9)
            for rr in range(2):
                row = r2 * 2 + rr
                for h in range(_H):
                    w = _lane(rec, rr * _H + h)
                    for cc in range(4):
                        o = row * _W1W + h * _C1 + cc * 16
                        accum[pl.ds(o, 16)] = accum[pl.ds(o, 16)] * w
            return 0
        lax.fori_loop(0, (_R1 + 1) // 2, drow, 0)
        pltpu.sync_copy(accum.at[pl.ds(0, _R1 * _W1W)],
                        out_h.at[pl.ds(lo * _W1W, _R1 * _W1W)])


def _sc_agg1(src, dst, ee1, feat1):
    f = pl.kernel(
        _agg1_body,
        out_type=jax.ShapeDtypeStruct((_NP1 * _W1W,), jnp.float32),
        mesh=_mesh(),
        compiler_params=pltpu.CompilerParams(needs_layout_passes=False),
        scratch_types=[
            pltpu.VMEM((_BE,), jnp.int32),
            pltpu.VMEM((_BE,), jnp.int32),
            pltpu.VMEM((_BE * _H,), jnp.float32),
            pltpu.VMEM((_BE + 16,), jnp.int32),
            pltpu.VMEM((_BE + 16,), jnp.int32),
            pltpu.VMEM((_BE + 16,), jnp.int32),
            pltpu.VMEM((16, _W1W), jnp.float32),
            pltpu.VMEM(((_R1 + 1) * _W1W,), jnp.float32),
            pltpu.VMEM(((_R1 + 1) * _H, ), jnp.float32),
            pltpu.SemaphoreType.DMA,
        ],
    )
    return f(src, dst, ee1, feat1)


_BE2 = 4000


def _agg2_body(src_h, dst_h, ee_h, feat_h, out_h,
               srcb0, dstb0, eeb0, srcb1, dstb1, eeb1, msrc, mdst, meid,
               fbuf, accum, den, sem, sem2):
    wid = _wid()
    srcbs = (srcb0, srcb1)
    dstbs = (dstb0, dstb1)
    eebs = (eeb0, eeb1)
    iota = lax.iota(jnp.int32, 16)
    zero16 = jnp.zeros((16,), jnp.float32)
    lo = wid * _R2
    nrow_pad = 320

    def zrow(r, _):
        for cc in range(_C2 // 16):
            accum[pl.ds(r * _C2 + cc * 16, 16)] = zero16
        return 0
    lax.fori_loop(0, nrow_pad, zrow, 0)

    def zden(i, _):
        den[pl.ds(i * 16, 16)] = zero16
        return 0
    lax.fori_loop(0, nrow_pad // 16, zden, 0)

    pltpu.async_copy(src_h.at[pl.ds(0, _BE2)], srcbs[0], sem2)
    pltpu.async_copy(dst_h.at[pl.ds(0, _BE2)], dstbs[0], sem2)
    pltpu.async_copy(ee_h.at[pl.ds(0, _BE2)], eebs[0], sem2)

    def blk2(b2, _):
      for u in range(2):
        b = b2 * 2 + u
        eb = b * _BE2
        srcb = srcbs[u]
        dstb = dstbs[u]
        eeblk = eebs[u]
        pltpu.make_async_copy(src_h.at[pl.ds(0, _BE2)], srcb, sem2).wait()
        pltpu.make_async_copy(dst_h.at[pl.ds(0, _BE2)], dstb, sem2).wait()
        pltpu.make_async_copy(ee_h.at[pl.ds(0, _BE2)], eeblk, sem2).wait()

        @pl.when(b + 1 < _E // _BE2)
        def _(eb=eb, u=u):
            pltpu.async_copy(src_h.at[pl.ds(eb + _BE2, _BE2)],
                             srcbs[1 - u], sem2)
            pltpu.async_copy(dst_h.at[pl.ds(eb + _BE2, _BE2)],
                             dstbs[1 - u], sem2)
            pltpu.async_copy(ee_h.at[pl.ds(eb + _BE2, _BE2)],
                             eebs[1 - u], sem2)

        def grp(g, cur, srcb=srcb, dstb=dstb):
            off = g * 16
            s16 = srcb[pl.ds(off, 16)]
            d16 = dstb[pl.ds(off, 16)]
            dr = d16 - lo
            m = (dr >= 0) & (dr < _R2)
            plsc.store_compressed(msrc.at[pl.ds(cur, 16)], s16, mask=m)
            plsc.store_compressed(mdst.at[pl.ds(cur, 16)], dr, mask=m)
            plsc.store_compressed(meid.at[pl.ds(cur, 16)], off + iota, mask=m)
            return cur + jnp.max(plsc.all_reduce_population_count(m))
        k = lax.fori_loop(0, _BE2 // 16, grp, jnp.int32(0))
        msrc[pl.ds(k, 16)] = jnp.zeros((16,), jnp.int32)
        mdst[pl.ds(k, 16)] = jnp.full((16,), _R2, jnp.int32)
        meid[pl.ds(k, 16)] = jnp.zeros((16,), jnp.int32)
        ng = (k + 15) // 16

        def proc(j, _, eeblk=eeblk):
            jo = j * 16
            ne = jnp.minimum(16, k - jo)
            pltpu.async_copy(feat_h.at[msrc.at[pl.ds(jo, 16)]], fbuf,
                             sem).wait()
            mei = meid[pl.ds(jo, 16)]
            md = mdst[pl.ds(jo, 16)]
            w16 = plsc.load_gather(eeblk, [mei])

            def edge(g2, _2):
                w = _lane(w16, g2)
                db = _lane(md, g2)
                plsc.addupdate_scatter(den, [db], w, mask=iota == 0)
                dbase = db * _C2
                for cc in range(4):
                    o = cc * 16
                    v = fbuf[g2, pl.ds(o, 16)]
                    plsc.addupdate_scatter(accum, [dbase + o + iota], w * v)
                return 0
            lax.fori_loop(0, ne, edge, 0)
            return 0
        lax.fori_loop(0, ng, proc, 0)
      return 0
    lax.fori_loop(0, _E // _BE2 // 2, blk2, 0)

    def drow(r16, _):
        d16 = den[pl.ds(r16 * 16, 16)]
        rec = 1.0 / (d16 + 1e-9)
        for rr in range(16):
            row = r16 * 16 + rr
            w = _lane(rec, rr)
            for cc in range(4):
                o = row * _C2 + cc * 16
                accum[pl.ds(o, 16)] = accum[pl.ds(o, 16)] * w
        return 0
    lax.fori_loop(0, nrow_pad // 16, drow, 0)
    pltpu.sync_copy(accum.at[pl.ds(0, _R2 * _C2)],
                    out_h.at[pl.ds(lo * _C2, _R2 * _C2)])


def _sc_agg2(src, dst, ee2, feat2):
    f = pl.kernel(
        _agg2_body,
        out_type=jax.ShapeDtypeStruct((_NP2 * _C2,), jnp.float32),
        mesh=_mesh(),
        compiler_params=pltpu.CompilerParams(needs_layout_passes=False),
        scratch_types=[
            pltpu.VMEM((_BE2,), jnp.int32),
            pltpu.VMEM((_BE2,), jnp.int32),
            pltpu.VMEM((_BE2,), jnp.float32),
            pltpu.VMEM((_BE2,), jnp.int32),
            pltpu.VMEM((_BE2,), jnp.int32),
            pltpu.VMEM((_BE2,), jnp.float32),
            pltpu.VMEM((_BE2 + 16,), jnp.int32),
            pltpu.VMEM((_BE2 + 16,), jnp.int32),
            pltpu.VMEM((_BE2 + 16,), jnp.int32),
            pltpu.VMEM((16, 128), jnp.float32),
            pltpu.VMEM((320 * _C2,), jnp.float32),
            pltpu.VMEM((320,), jnp.float32),
            pltpu.SemaphoreType.DMA,
            pltpu.SemaphoreType.DMA,
        ],
    )
    return f(src, dst, ee2, feat2)


_BM = 400


def _tca_body(x_ref, w_ref, a_ref, f_ref, aux_ref):
    f = jnp.dot(x_ref[...], w_ref[...], preferred_element_type=jnp.float32)
    f_ref[...] = f
    aux_ref[...] = jnp.dot(f, a_ref[...], preferred_element_type=jnp.float32)


def _tc_a(x, w1r, acmb):
    return pl.pallas_call(
        _tca_body,
        grid=(_N // _BM,),
        in_specs=[pl.BlockSpec((_BM, _D), lambda i: (i, 0)),
                  pl.BlockSpec((_D, _W1W), lambda i: (0, 0)),
                  pl.BlockSpec((_W1W, 128), lambda i: (0, 0))],
        out_specs=[pl.BlockSpec((_BM, _W1W), lambda i: (i, 0)),
                   pl.BlockSpec((_BM, 128), lambda i: (i, 0))],
        out_shape=[jax.ShapeDtypeStruct((_N, _W1W), jnp.float32),
                   jax.ShapeDtypeStruct((_N, 128), jnp.float32)],
    )(x, w1r, acmb)


def _tcb_body(o1_ref, b1_ref, w2_ref, a2_ref, f2_ref, aux2_ref):
    v = o1_ref[...] + b1_ref[...]
    h1 = jnp.where(v > 0.0, v, jnp.exp(v) - 1.0)
    f2 = jnp.dot(h1, w2_ref[...], preferred_element_type=jnp.float32)
    f2_ref[...] = f2
    aux2_ref[...] = jnp.dot(f2, a2_ref[...],
                            preferred_element_type=jnp.float32)


def _tc_b(o1, b1r, w2r, a2cmb):
    return pl.pallas_call(
        _tcb_body,
        grid=(_N // _BM,),
        in_specs=[pl.BlockSpec((_BM, _W1W), lambda i: (i, 0)),
                  pl.BlockSpec((1, _W1W), lambda i: (0, 0)),
                  pl.BlockSpec((_W1W, 128), lambda i: (0, 0)),
                  pl.BlockSpec((128, 128), lambda i: (0, 0))],
        out_specs=[pl.BlockSpec((_BM, 128), lambda i: (i, 0)),
                   pl.BlockSpec((_BM, 128), lambda i: (i, 0))],
        out_shape=[jax.ShapeDtypeStruct((_N, 128), jnp.float32),
                   jax.ShapeDtypeStruct((_N, 128), jnp.float32)],
    )(o1, b1r, w2r, a2cmb)


def _tcc_body(o2_ref, b2_ref, wd_ref, bd_ref, g_ref, l_ref):
    i = pl.program_id(0)
    v = o2_ref[...] + b2_ref[...]
    h2 = jnp.where(v > 0.0, v, jnp.exp(v) - 1.0)
    ps = jnp.sum(h2, axis=0, keepdims=True)

    @pl.when(i == 0)
    def _():
        g_ref[...] = ps

    @pl.when(i > 0)
    def _():
        g_ref[...] = g_ref[...] + ps

    @pl.when(i == pl.num_programs(0) - 1)
    def _():
        l = jnp.dot(g_ref[...], wd_ref[...],
                    preferred_element_type=jnp.float32) + bd_ref[...]
        m = jnp.max(l, axis=-1, keepdims=True)
        z = jnp.exp(l - m)
        l_ref[...] = z / jnp.sum(z, axis=-1, keepdims=True)


def _tc_c(o2, b2r, wdp, bdp):
    bm = 400
    return pl.pallas_call(
        _tcc_body,
        grid=(_N // bm,),
        in_specs=[pl.BlockSpec((bm, _C2), lambda i: (i, 0)),
                  pl.BlockSpec((1, _C2), lambda i: (0, 0)),
                  pl.BlockSpec((_C2, 128), lambda i: (0, 0)),
                  pl.BlockSpec((1, 128), lambda i: (0, 0))],
        out_specs=[pl.BlockSpec((1, _C2), lambda i: (0, 0)),
                   pl.BlockSpec((1, 128), lambda i: (0, 0))],
        out_shape=[jax.ShapeDtypeStruct((1, _C2), jnp.float32),
                   jax.ShapeDtypeStruct((1, 128), jnp.float32)],
    )(o2, b2r, wdp, bdp)


def _pack(a_hi, a_lo):
    hi = lax.bitcast_convert_type(a_hi.astype(jnp.bfloat16),
                                  jnp.uint16).astype(jnp.uint32) << 16
    lo = lax.bitcast_convert_type(a_lo.astype(jnp.bfloat16),
                                  jnp.uint16).astype(jnp.uint32)
    return lax.bitcast_convert_type(hi | lo, jnp.int32)


def kernel(x, edge_index, W1, a_src1, a_dst1, b1, W2, a_src2, a_dst2, b2,
           Wd, bd):
    src = edge_index[0]
    dst = edge_index[1]
    w1r = W1.reshape(_D, _W1W)
    eye8 = jnp.eye(_H, dtype=jnp.float32)
    acmb = jnp.concatenate([
        jnp.einsum("hc,hk->hck", a_src1, eye8).reshape(_W1W, _H),
        jnp.einsum("hc,hk->hck", a_dst1, eye8).reshape(_W1W, _H),
    ], axis=1)
    acmb = jnp.pad(acmb, ((0, 0), (0, 128 - 2 * _H)))
    w2r = jnp.pad(W2.reshape(_W1W, _C2), ((0, 0), (0, 128 - _C2)))
    a2cmb = jnp.pad(jnp.concatenate([a_src2.T, a_dst2.T], axis=1),
                    ((0, 128 - _C2), (0, 126)))
    wdp = jnp.pad(Wd, ((0, 0), (0, 128 - _NL)))
    bdp = jnp.concatenate(
        [bd, jnp.full((128 - _NL,), -1e30, jnp.float32)]).reshape(1, 128)

    feat1, aux1 = _tc_a(x, w1r, acmb)
    pk1 = _pack(aux1[:, :_H], aux1[:, _H:2 * _H]).reshape(-1)
    ee1 = _sc_ee1(pk1, src, dst)
    out1 = _sc_agg1(src, dst, ee1, feat1).reshape(_NP1, _W1W)[:_N]
    feat2, aux2 = _tc_b(out1, b1.reshape(1, _W1W), w2r, a2cmb)
    pk2 = _pack(aux2[:, 0:1], aux2[:, 1:2]).reshape(-1)
    ee2 = _sc_ee2(pk2, src, dst)
    out2 = _sc_agg2(src, dst, ee2, feat2).reshape(_NP2, _C2)[:_N]
    _, probs = _tc_c(out2, b2.reshape(1, _C2), wdp, bdp)
    return probs[:, :_NL]
